# baseline (speedup 1.0000x reference)
import jax
import jax.numpy as jnp
from jax.experimental import pallas as pl
from jax.experimental.pallas import tpu as pltpu


ROWS = 256
NBUF = 16
LEAD = 4


def _bcast_copy_kernel(w_hbm, out_hbm, buf, rsem, wsem):
    batch = out_hbm.shape[0]
    seq = w_hbm.shape[0]
    nchunk = seq // ROWS

    def read(c):
        s = c % NBUF
        return pltpu.make_async_copy(
            w_hbm.at[pl.ds(c * ROWS, ROWS), :], buf.at[s], rsem.at[s])

    def write(c, b):
        s = c % NBUF
        return pltpu.make_async_copy(
            buf.at[s], out_hbm.at[b, pl.ds(c * ROWS, ROWS), :], wsem.at[s])

    for c in range(min(LEAD, nchunk)):
        read(c).start()
    for c in range(nchunk):
        read(c).wait()
        for b in range(batch):
            write(c, b).start()
        n = c + LEAD
        if n < nchunk:
            prev = n - NBUF
            if prev >= 0:
                for b in range(batch):
                    write(prev, b).wait()
            read(n).start()
    for c in range(max(0, nchunk - NBUF), nchunk):
        for b in range(batch):
            write(c, b).wait()


def kernel(tokens, W_pos):
    batch = tokens.shape[0]
    seq_len = tokens.shape[1]
    d = W_pos.shape[1]

    out = pl.pallas_call(
        _bcast_copy_kernel,
        in_specs=[pl.BlockSpec(memory_space=pltpu.MemorySpace.HBM)],
        out_specs=pl.BlockSpec(memory_space=pltpu.MemorySpace.HBM),
        out_shape=jax.ShapeDtypeStruct((batch, seq_len, d), W_pos.dtype),
        scratch_shapes=[
            pltpu.VMEM((NBUF, ROWS, d), W_pos.dtype),
            pltpu.SemaphoreType.DMA((NBUF,)),
            pltpu.SemaphoreType.DMA((NBUF,)),
        ],
    )(W_pos[:seq_len])
    return out


# speedup vs baseline: 1.0237x; 1.0237x over previous
import jax
import jax.numpy as jnp
from jax.experimental import pallas as pl
from jax.experimental.pallas import tpu as pltpu


ROWS = 1024
NBUF = 6
LEAD = 2


def _bcast_copy_kernel(w_hbm, out_hbm, buf, rsem, wsem):
    batch = out_hbm.shape[0]
    seq = w_hbm.shape[0]
    nchunk = seq // ROWS

    def read(c):
        s = c % NBUF
        return pltpu.make_async_copy(
            w_hbm.at[pl.ds(c * ROWS, ROWS), :], buf.at[s], rsem.at[s])

    def write(c, b):
        s = c % NBUF
        return pltpu.make_async_copy(
            buf.at[s], out_hbm.at[b, pl.ds(c * ROWS, ROWS), :], wsem.at[s])

    for c in range(min(LEAD, nchunk)):
        read(c).start()
    for c in range(nchunk):
        read(c).wait()
        for b in range(batch):
            write(c, b).start()
        n = c + LEAD
        if n < nchunk:
            prev = n - NBUF
            if prev >= 0:
                for b in range(batch):
                    write(prev, b).wait()
            read(n).start()
    for c in range(max(0, nchunk - NBUF), nchunk):
        for b in range(batch):
            write(c, b).wait()


def kernel(tokens, W_pos):
    batch = tokens.shape[0]
    seq_len = tokens.shape[1]
    d = W_pos.shape[1]

    out = pl.pallas_call(
        _bcast_copy_kernel,
        in_specs=[pl.BlockSpec(memory_space=pltpu.MemorySpace.HBM)],
        out_specs=pl.BlockSpec(memory_space=pltpu.MemorySpace.HBM),
        out_shape=jax.ShapeDtypeStruct((batch, seq_len, d), W_pos.dtype),
        scratch_shapes=[
            pltpu.VMEM((NBUF, ROWS, d), W_pos.dtype),
            pltpu.SemaphoreType.DMA((NBUF,)),
            pltpu.SemaphoreType.DMA((NBUF,)),
        ],
    )(W_pos[:seq_len])
    return out
